# Initial kernel scaffold; baseline (speedup 1.0000x reference)
import jax
import jax.numpy as jnp
from jax.experimental import pallas as pl


def kernel(x, edge_index, W1, a_src1, a_dst1, b1, W2, a_src2, a_dst2, b2, Wf, bf, Ws, bs):
    raise NotImplementedError("write your pallas kernel here")


# speedup vs baseline: 15.3626x; 15.3626x over previous
import functools

import jax
import jax.numpy as jnp
from jax import lax
from jax.experimental import pallas as pl
from jax.experimental.pallas import tpu as pltpu
from jax.experimental.pallas import tpu_sc as plsc

N = 10000
NP = 10240
D = 128
DA = 136
E = 320000
ET = E + N
EROWS = 2624
EP = EROWS * 128
NW = 32
WROWS = EROWS // NW
RPT = NP // 16

_mesh = plsc.VectorSubcoreMesh(core_axis_name="c", subcore_axis_name="s")


def _mm_first_body(x_ref, w_ref, h_ref, al_ref):
    h = jnp.dot(x_ref[...], w_ref[...], preferred_element_type=jnp.float32)
    h_ref[...] = h[:, :D]
    al_ref[...] = h[:, D:D + 2]


def _finish_layer(nd_ref, den_ref, b_ref):
    t = nd_ref[0] + nd_ref[1]
    ones = jnp.ones((NW, 1), jnp.float32)
    dsum = lax.dot_general(den_ref[...], ones, (((0,), (0,)), ((), ())),
                           preferred_element_type=jnp.float32)
    den = jnp.maximum(dsum, 1e-30)
    return jnp.maximum(t / den + b_ref[...], 0.0)


def _mm_mid_body(nd_ref, den_ref, b_ref, w_ref, h_ref, al_ref):
    x2 = _finish_layer(nd_ref, den_ref, b_ref)
    h = jnp.dot(x2, w_ref[...], preferred_element_type=jnp.float32)
    h_ref[...] = h[:, :D]
    al_ref[...] = h[:, D:D + 2]


def _mm_last_body(nd_ref, den_ref, b_ref, w_ref, bo_ref, o_ref):
    x3 = _finish_layer(nd_ref, den_ref, b_ref)
    o_ref[...] = (
        jnp.dot(x3, w_ref[...], preferred_element_type=jnp.float32)
        + bo_ref[...]
    )


def _mm_first(xp, wp):
    return pl.pallas_call(
        _mm_first_body,
        grid=(NP // 512,),
        in_specs=[
            pl.BlockSpec((512, D), lambda i: (i, 0)),
            pl.BlockSpec((D, DA), lambda i: (0, 0)),
        ],
        out_specs=[
            pl.BlockSpec((512, D), lambda i: (i, 0)),
            pl.BlockSpec((512, 2), lambda i: (i, 0)),
        ],
        out_shape=[
            jax.ShapeDtypeStruct((NP, D), jnp.float32),
            jax.ShapeDtypeStruct((NP, 2), jnp.float32),
        ],
    )(xp, wp)


def _mm_mid(nd, den, b, wp):
    return pl.pallas_call(
        _mm_mid_body,
        grid=(NP // 512,),
        in_specs=[
            pl.BlockSpec((2, 512, D), lambda i: (0, i, 0)),
            pl.BlockSpec((NW, 512), lambda i: (0, i)),
            pl.BlockSpec((1, D), lambda i: (0, 0)),
            pl.BlockSpec((D, DA), lambda i: (0, 0)),
        ],
        out_specs=[
            pl.BlockSpec((512, D), lambda i: (i, 0)),
            pl.BlockSpec((512, 2), lambda i: (i, 0)),
        ],
        out_shape=[
            jax.ShapeDtypeStruct((NP, D), jnp.float32),
            jax.ShapeDtypeStruct((NP, 2), jnp.float32),
        ],
    )(nd, den, b, wp)


def _mm_last(nd, den, b, wo, bo):
    return pl.pallas_call(
        _mm_last_body,
        grid=(NP // 512,),
        in_specs=[
            pl.BlockSpec((2, 512, D), lambda i: (0, i, 0)),
            pl.BlockSpec((NW, 512), lambda i: (0, i)),
            pl.BlockSpec((1, D), lambda i: (0, 0)),
            pl.BlockSpec((D, 16), lambda i: (0, 0)),
            pl.BlockSpec((1, 16), lambda i: (0, 0)),
        ],
        out_specs=pl.BlockSpec((512, 16), lambda i: (i, 0)),
        out_shape=jax.ShapeDtypeStruct((NP, 16), jnp.float32),
    )(nd, den, b, wo, bo)


@functools.partial(
    pl.kernel,
    out_type=[
        jax.ShapeDtypeStruct((2, NP, D), jnp.float32),
        jax.ShapeDtypeStruct((NW, NP), jnp.float32),
    ],
    mesh=_mesh,
    compiler_params=pltpu.CompilerParams(
        needs_layout_passes=False, use_tc_tiling_on_sc=False),
    scratch_types=[
        pltpu.VMEM((2 * NP,), jnp.float32),
        pltpu.VMEM((NP + 16,), jnp.float32),
        pltpu.VMEM((1, 128), jnp.int32),
        pltpu.VMEM((1, 128), jnp.int32),
        pltpu.VMEM((128, D), jnp.float32),
        pltpu.VMEM_SHARED((NP, D), jnp.float32),
        pltpu.SemaphoreType.DMA,
        pltpu.SemaphoreType.DMA,
    ],
)
def _sc_edge(al_hbm, src_hbm, dst_hbm, h_hbm, z_hbm, num_out, den_out,
             al_v, den_t, srcb, dstb, rows, num_sp, gsem, ssem):
    c = lax.axis_index("c")
    s = lax.axis_index("s")
    wid = s * 2 + c
    iota16 = lax.iota(jnp.int32, 16)
    zf16 = jnp.zeros((16,), jnp.float32)

    pltpu.sync_copy(al_hbm, al_v)
    pltpu.sync_copy(z_hbm.at[pl.ds(s * RPT, RPT)],
                    num_sp.at[pl.ds(s * RPT, RPT)])

    def zden_body(i, carry):
        den_t[pl.ds(i * 16, 16)] = zf16
        return carry

    lax.fori_loop(0, (NP + 16) // 16, zden_body, 0)
    plsc.subcore_barrier()

    def chunk_body(ch, carry):
        rowbase = wid * WROWS + ch
        pltpu.sync_copy(src_hbm.at[pl.ds(rowbase, 1)], srcb)
        pltpu.sync_copy(dst_hbm.at[pl.ds(rowbase, 1)], dstb)
        cp = pltpu.async_copy(h_hbm.at[srcb.at[0]], rows, gsem)
        cp.wait()

        def group_body(g, carry2):
            srcv = srcb[0, pl.ds(g * 16, 16)]
            dstv = dstb[0, pl.ds(g * 16, 16)]
            dstv2 = dstv * 2
            as_s = plsc.load_gather(al_v, [srcv * 2])
            as_d = plsc.load_gather(al_v, [dstv2])
            ad_d = plsc.load_gather(al_v, [dstv2 + 1])
            e = as_s + ad_d
            e = jnp.maximum(e, 0.2 * e)
            m = as_d + ad_d
            m = jnp.maximum(m, 0.2 * m)
            exv = jnp.exp(e - m)
            for l in range(16):
                i = g * 16 + l
                exb = jnp.full((16,), exv[l], jnp.float32)
                for f in range(8):
                    rows[i, pl.ds(f * 16, 16)] = (
                        rows[i, pl.ds(f * 16, 16)] * exb)
                d = dstv[l]
                win = den_t[pl.ds(d, 16)]
                den_t[pl.ds(d, 16)] = win + jnp.where(iota16 == 0, exb, 0.0)
            return carry2

        lax.fori_loop(0, 8, group_body, 0)
        sc = pltpu.async_copy(rows, num_sp.at[dstb.at[0]], ssem, add=True)
        sc.wait()
        return carry

    lax.fori_loop(0, WROWS, chunk_body, 0)
    plsc.subcore_barrier()
    pltpu.sync_copy(num_sp.at[pl.ds(s * RPT, RPT)],
                    num_out.at[c, pl.ds(s * RPT, RPT)])
    pltpu.sync_copy(den_t.at[pl.ds(0, NP)], den_out.at[wid])


def kernel(x, edge_index, W1, a_src1, a_dst1, b1, W2, a_src2, a_dst2, b2,
           Wf, bf, Ws, bs):
    f32 = jnp.float32
    xp = jnp.zeros((NP, D), f32).at[:N].set(x)

    def augment(W, a_src, a_dst):
        return jnp.concatenate(
            [W, (W @ a_src)[:, None], (W @ a_dst)[:, None],
             jnp.zeros((D, DA - D - 2), f32)], axis=1)

    w1p = augment(W1, a_src1, a_dst1)
    w2p = augment(W2, a_src2, a_dst2)
    wo = jnp.concatenate([Wf, Ws, jnp.zeros((D, 6), f32)], axis=1)
    bo = jnp.concatenate([bf, bs, jnp.zeros((6,), f32)])[None, :]

    sl = jnp.arange(N, dtype=jnp.int32)
    pad = jnp.full((EP - ET,), N, jnp.int32)
    src2d = jnp.concatenate([edge_index[0], sl, pad]).reshape(EROWS, 128)
    dst2d = jnp.concatenate([edge_index[1], sl, pad]).reshape(EROWS, 128)
    znd = jnp.zeros((NP, D), f32)

    h1, al1 = _mm_first(xp, w1p)
    nd1, den1 = _sc_edge(al1.reshape(2 * NP), src2d, dst2d, h1, znd)
    h2, al2 = _mm_mid(nd1, den1, b1[None, :], w2p)
    nd2, den2 = _sc_edge(al2.reshape(2 * NP), src2d, dst2d, h2, znd)
    out = _mm_last(nd2, den2, b2[None, :], wo, bo)
    return (out[:N, :3], out[:N, 3:10])


# speedup vs baseline: 17.8865x; 1.1643x over previous
import functools

import jax
import jax.numpy as jnp
from jax import lax
from jax.experimental import pallas as pl
from jax.experimental.pallas import tpu as pltpu
from jax.experimental.pallas import tpu_sc as plsc

N = 10000
NP = 10240
D = 128
DA = 136
E = 320000
ET = E + N
EROWS = 5248
CH = 64
EP = EROWS * CH
NW = 32
WROWS = EROWS // NW
NG = CH // 16
RPT = NP // 16

_mesh = plsc.VectorSubcoreMesh(core_axis_name="c", subcore_axis_name="s")


def _mm_first_body(x_ref, w_ref, h_ref, al_ref):
    h = jnp.dot(x_ref[...], w_ref[...], preferred_element_type=jnp.float32)
    h_ref[...] = h[:, :D]
    al_ref[...] = h[:, D:D + 2]


def _finish_layer(nd_ref, den_ref, b_ref):
    t = nd_ref[0] + nd_ref[1]
    ones = jnp.ones((NW, 1), jnp.float32)
    dsum = lax.dot_general(den_ref[...], ones, (((0,), (0,)), ((), ())),
                           preferred_element_type=jnp.float32)
    den = jnp.maximum(dsum, 1e-30)
    return jnp.maximum(t / den + b_ref[...], 0.0)


def _mm_mid_body(nd_ref, den_ref, b_ref, w_ref, h_ref, al_ref):
    x2 = _finish_layer(nd_ref, den_ref, b_ref)
    h = jnp.dot(x2, w_ref[...], preferred_element_type=jnp.float32)
    h_ref[...] = h[:, :D]
    al_ref[...] = h[:, D:D + 2]


def _mm_last_body(nd_ref, den_ref, b_ref, w_ref, bo_ref, o_ref):
    x3 = _finish_layer(nd_ref, den_ref, b_ref)
    o_ref[...] = (
        jnp.dot(x3, w_ref[...], preferred_element_type=jnp.float32)
        + bo_ref[...]
    )


def _mm_first(xp, wp):
    return pl.pallas_call(
        _mm_first_body,
        grid=(NP // 512,),
        in_specs=[
            pl.BlockSpec((512, D), lambda i: (i, 0)),
            pl.BlockSpec((D, DA), lambda i: (0, 0)),
        ],
        out_specs=[
            pl.BlockSpec((512, D), lambda i: (i, 0)),
            pl.BlockSpec((512, 2), lambda i: (i, 0)),
        ],
        out_shape=[
            jax.ShapeDtypeStruct((NP, D), jnp.float32),
            jax.ShapeDtypeStruct((NP, 2), jnp.float32),
        ],
    )(xp, wp)


def _mm_mid(nd, den, b, wp):
    return pl.pallas_call(
        _mm_mid_body,
        grid=(NP // 512,),
        in_specs=[
            pl.BlockSpec((2, 512, D), lambda i: (0, i, 0)),
            pl.BlockSpec((NW, 512), lambda i: (0, i)),
            pl.BlockSpec((1, D), lambda i: (0, 0)),
            pl.BlockSpec((D, DA), lambda i: (0, 0)),
        ],
        out_specs=[
            pl.BlockSpec((512, D), lambda i: (i, 0)),
            pl.BlockSpec((512, 2), lambda i: (i, 0)),
        ],
        out_shape=[
            jax.ShapeDtypeStruct((NP, D), jnp.float32),
            jax.ShapeDtypeStruct((NP, 2), jnp.float32),
        ],
    )(nd, den, b, wp)


def _mm_last(nd, den, b, wo, bo):
    return pl.pallas_call(
        _mm_last_body,
        grid=(NP // 512,),
        in_specs=[
            pl.BlockSpec((2, 512, D), lambda i: (0, i, 0)),
            pl.BlockSpec((NW, 512), lambda i: (0, i)),
            pl.BlockSpec((1, D), lambda i: (0, 0)),
            pl.BlockSpec((D, 16), lambda i: (0, 0)),
            pl.BlockSpec((1, 16), lambda i: (0, 0)),
        ],
        out_specs=pl.BlockSpec((512, 16), lambda i: (i, 0)),
        out_shape=jax.ShapeDtypeStruct((NP, 16), jnp.float32),
    )(nd, den, b, wo, bo)


@functools.partial(
    pl.kernel,
    out_type=[
        jax.ShapeDtypeStruct((2, NP, D), jnp.float32),
        jax.ShapeDtypeStruct((NW, NP), jnp.float32),
    ],
    mesh=_mesh,
    compiler_params=pltpu.CompilerParams(
        needs_layout_passes=False, use_tc_tiling_on_sc=False),
    scratch_types=[
        pltpu.VMEM((2 * NP,), jnp.float32),
        pltpu.VMEM((NP + 16,), jnp.float32),
        pltpu.VMEM((1, CH), jnp.int32),
        pltpu.VMEM((1, CH), jnp.int32),
        pltpu.VMEM((1, CH), jnp.int32),
        pltpu.VMEM((CH, D), jnp.float32),
        pltpu.VMEM((CH, D), jnp.float32),
        pltpu.VMEM((CH,), jnp.float32),
        pltpu.VMEM_SHARED((NP, D), jnp.float32),
        pltpu.SemaphoreType.DMA,
        pltpu.SemaphoreType.DMA,
        pltpu.SemaphoreType.DMA,
        pltpu.SemaphoreType.DMA,
    ],
)
def _sc_edge(al_hbm, src_hbm, dst_hbm, h_hbm, z_hbm, num_out, den_out,
             al_v, den_t, srcb, dstb_e, dstb_o, rows_e, rows_o, exb,
             num_sp, gsem_e, gsem_o, ssem_e, ssem_o):
    c = lax.axis_index("c")
    s = lax.axis_index("s")
    wid = s * 2 + c
    base = wid * WROWS
    zf16 = jnp.zeros((16,), jnp.float32)

    pltpu.sync_copy(al_hbm, al_v)
    pltpu.sync_copy(z_hbm.at[pl.ds(s * RPT, RPT)],
                    num_sp.at[pl.ds(s * RPT, RPT)])

    def zden_body(i, carry):
        den_t[pl.ds(i * 16, 16)] = zf16
        return carry

    lax.fori_loop(0, (NP + 16) // 16, zden_body, 0)
    plsc.subcore_barrier()

    pltpu.sync_copy(src_hbm.at[pl.ds(base, 1)], srcb)
    pltpu.sync_copy(dst_hbm.at[pl.ds(base, 1)], dstb_e)
    pltpu.async_copy(h_hbm.at[srcb.at[0]], rows_e, gsem_e)

    def _sub_iter(cur, dstb_x, rows_x, gsem_x, ssem_x,
                  dstb_y, rows_y, ssem_y, gsem_y, wait_y, do_pref):
        def ex_body(g, carry2):
            srcv = srcb[0, pl.ds(g * 16, 16)]
            dstv = dstb_x[0, pl.ds(g * 16, 16)]
            dstv2 = dstv * 2
            as_s = plsc.load_gather(al_v, [srcv * 2])
            as_d = plsc.load_gather(al_v, [dstv2])
            ad_d = plsc.load_gather(al_v, [dstv2 + 1])
            e = as_s + ad_d
            e = jnp.maximum(e, 0.2 * e)
            m = as_d + ad_d
            m = jnp.maximum(m, 0.2 * m)
            exv = jnp.exp(e - m)
            exb[pl.ds(g * 16, 16)] = exv
            plsc.addupdate_scatter(den_t, [dstv], exv)
            return carry2

        lax.fori_loop(0, NG, ex_body, 0)
        pltpu.make_async_copy(h_hbm.at[srcb.at[0]], rows_x, gsem_x).wait()

        @pl.when(wait_y)
        def _():
            pltpu.make_async_copy(
                rows_y, num_sp.at[dstb_y.at[0]], ssem_y).wait()

        @pl.when(do_pref)
        def _():
            pltpu.sync_copy(src_hbm.at[pl.ds(cur + 1, 1)], srcb)
            pltpu.sync_copy(dst_hbm.at[pl.ds(cur + 1, 1)], dstb_y)
            pltpu.async_copy(h_hbm.at[srcb.at[0]], rows_y, gsem_y)

        def sc_body(g, carry2):
            exv16 = exb[pl.ds(g * 16, 16)]
            for l in range(16):
                i = g * 16 + l
                exq = jnp.full((16,), exv16[l], jnp.float32)
                for f in range(8):
                    rows_x[i, pl.ds(f * 16, 16)] = (
                        rows_x[i, pl.ds(f * 16, 16)] * exq)
            return carry2

        lax.fori_loop(0, NG, sc_body, 0)
        pltpu.async_copy(rows_x, num_sp.at[dstb_x.at[0]], ssem_x, add=True)

    def pair_body(p, carry):
        ce = base + 2 * p
        _sub_iter(ce, dstb_e, rows_e, gsem_e, ssem_e,
                  dstb_o, rows_o, ssem_o, gsem_o, p > 0, p >= 0)
        _sub_iter(ce + 1, dstb_o, rows_o, gsem_o, ssem_o,
                  dstb_e, rows_e, ssem_e, gsem_e, p >= 0,
                  p < WROWS // 2 - 1)
        return carry

    lax.fori_loop(0, WROWS // 2, pair_body, 0)
    pltpu.make_async_copy(rows_o, num_sp.at[dstb_o.at[0]], ssem_o).wait()
    plsc.subcore_barrier()
    pltpu.sync_copy(num_sp.at[pl.ds(s * RPT, RPT)],
                    num_out.at[c, pl.ds(s * RPT, RPT)])
    pltpu.sync_copy(den_t.at[pl.ds(0, NP)], den_out.at[wid])


def kernel(x, edge_index, W1, a_src1, a_dst1, b1, W2, a_src2, a_dst2, b2,
           Wf, bf, Ws, bs):
    f32 = jnp.float32
    xp = jnp.zeros((NP, D), f32).at[:N].set(x)

    def augment(W, a_src, a_dst):
        return jnp.concatenate(
            [W, (W @ a_src)[:, None], (W @ a_dst)[:, None],
             jnp.zeros((D, DA - D - 2), f32)], axis=1)

    w1p = augment(W1, a_src1, a_dst1)
    w2p = augment(W2, a_src2, a_dst2)
    wo = jnp.concatenate([Wf, Ws, jnp.zeros((D, 6), f32)], axis=1)
    bo = jnp.concatenate([bf, bs, jnp.zeros((6,), f32)])[None, :]

    sl = jnp.arange(N, dtype=jnp.int32)
    pad = jnp.full((EP - ET,), N, jnp.int32)
    src2d = jnp.concatenate([edge_index[0], sl, pad]).reshape(EROWS, CH)
    dst2d = jnp.concatenate([edge_index[1], sl, pad]).reshape(EROWS, CH)
    znd = jnp.zeros((NP, D), f32)

    h1, al1 = _mm_first(xp, w1p)
    nd1, den1 = _sc_edge(al1.reshape(2 * NP), src2d, dst2d, h1, znd)
    h2, al2 = _mm_mid(nd1, den1, b1[None, :], w2p)
    nd2, den2 = _sc_edge(al2.reshape(2 * NP), src2d, dst2d, h2, znd)
    out = _mm_last(nd2, den2, b2[None, :], wo, bo)
    return (out[:N, :3], out[:N, 3:10])
